# Initial kernel scaffold; baseline (speedup 1.0000x reference)
import jax
import jax.numpy as jnp
from jax.experimental import pallas as pl


def kernel(x, edge_index, W1, b1, gamma, beta, W2, b2, W3, b3):
    raise NotImplementedError("write your pallas kernel here")


# speedup vs baseline: 32.1209x; 32.1209x over previous
import functools

import numpy as np
import jax
import jax.numpy as jnp
from jax import lax
from jax.experimental import pallas as pl
from jax.experimental.pallas import tpu as pltpu
from jax.experimental.pallas import tpu_sc as plsc

N = 10000
E = 320000
D = 128
H1 = 256
H2 = 128
C = 6

NC, NS = 2, 16
NW = NC * NS
EPW = E // NW
B = 80
NB = EPW // B
NPAD = 10240
RPT = NPAD // NS
RCH = B
NCH = RPT // RCH

_mesh = plsc.VectorSubcoreMesh(core_axis_name="c", subcore_axis_name="s")


@functools.partial(
    pl.kernel,
    mesh=_mesh,
    out_type=jax.ShapeDtypeStruct((NW, N), jnp.float32),
    scratch_types=[
        pltpu.VMEM((EPW,), jnp.int32),
        pltpu.VMEM((N,), jnp.float32),
    ],
    compiler_params=pltpu.CompilerParams(needs_layout_passes=False),
)
def _deg_kernel(dst_hbm, degp_hbm, idx_v, deg_v):
    wid = lax.axis_index("s") * NC + lax.axis_index("c")

    def zero(i, _):
        deg_v[pl.ds(i * 16, 16)] = jnp.zeros((16,), jnp.float32)
        return 0

    lax.fori_loop(0, N // 16, zero, 0)
    pltpu.sync_copy(dst_hbm.at[pl.ds(wid * EPW, EPW)], idx_v)
    ones = jnp.ones((16,), jnp.float32)

    def count(k, _):
        iv = idx_v[pl.ds(k * 16, 16)]
        plsc.addupdate_scatter(deg_v, [iv], ones)
        return 0

    lax.fori_loop(0, EPW // 16, count, 0)
    pltpu.sync_copy(deg_v, degp_hbm.at[wid])


def _scale_body(degp_ref, x_ref, y_ref, dinv_ref):
    deg = jnp.sum(degp_ref[...], axis=1, keepdims=True) + 1.0
    dinv = lax.rsqrt(deg)
    dinv_ref[...] = dinv
    y_ref[...] = x_ref[...] * dinv


@functools.partial(
    pl.kernel,
    mesh=_mesh,
    out_type=jax.ShapeDtypeStruct((NC, NPAD, D), jnp.float32),
    scratch_types=[
        pltpu.VMEM((NB, B), jnp.int32),
        pltpu.VMEM((NB, B), jnp.int32),
        pltpu.VMEM((B, D), jnp.float32),
        pltpu.VMEM_SHARED((NPAD, D), jnp.float32),
        pltpu.SemaphoreType.DMA,
    ],
    compiler_params=pltpu.CompilerParams(needs_layout_passes=False),
)
def _agg_kernel(y_hbm, src3_hbm, dst3_hbm, zp_hbm,
                srcb_v, dstb_v, rows_v, zsh, sem):
    cid = lax.axis_index("c")
    sid = lax.axis_index("s")
    wid = cid * NS + sid
    row0 = sid * RPT

    def zbuf(k, _):
        rows_v[k // (D // 16), pl.ds((k % (D // 16)) * 16, 16)] = (
            jnp.zeros((16,), jnp.float32))
        return 0

    lax.fori_loop(0, RCH * D // 16, zbuf, 0)

    def zshared(i, _):
        pltpu.sync_copy(rows_v, zsh.at[pl.ds(row0 + i * RCH, RCH)])
        return 0

    lax.fori_loop(0, NCH, zshared, 0)
    pltpu.sync_copy(src3_hbm.at[wid], srcb_v)
    pltpu.sync_copy(dst3_hbm.at[wid], dstb_v)
    plsc.subcore_barrier()

    def edge_batch(j, _):
        pltpu.async_copy(y_hbm.at[srcb_v.at[j]], rows_v, sem).wait()
        pltpu.sync_copy(rows_v, zsh.at[dstb_v.at[j]], add=True)
        return 0

    lax.fori_loop(0, NB, edge_batch, 0)
    plsc.subcore_barrier()

    def writeback(i, _):
        sl = pl.ds(row0 + i * RCH, RCH)
        pltpu.sync_copy(zsh.at[sl], rows_v)
        pltpu.sync_copy(rows_v, zp_hbm.at[cid, sl])
        return 0

    lax.fori_loop(0, NCH, writeback, 0)


_BN_SCALE = float(1.0 / np.sqrt(1.0 + 1e-5))


def _mlp_body(zp_ref, y_ref, dinv_ref, W1_ref, b1_ref, g_ref, be_ref,
              W2_ref, b2_ref, W3_ref, b3_ref, out_ref):
    agg = (zp_ref[0] + zp_ref[1] + y_ref[...]) * dinv_ref[...]
    h = jnp.dot(agg, W1_ref[...], preferred_element_type=jnp.float32)
    h = jnp.maximum(h + b1_ref[...], 0.0)
    h = h * (g_ref[...] * _BN_SCALE) + be_ref[...]
    h = jnp.dot(h, W2_ref[...], preferred_element_type=jnp.float32)
    h = jnp.maximum(h + b2_ref[...], 0.0)
    out_ref[...] = (
        jnp.dot(h, W3_ref[...], preferred_element_type=jnp.float32)
        + b3_ref[...])


def kernel(x, edge_index, W1, b1, gamma, beta, W2, b2, W3, b3):
    src = edge_index[0]
    dst = edge_index[1]
    src3 = src.reshape(NW, NB, B)
    dst3 = dst.reshape(NW, NB, B)

    degp = _deg_kernel(dst)
    degp_t = degp.T

    y, dinv = pl.pallas_call(
        _scale_body,
        out_shape=[
            jax.ShapeDtypeStruct((N, D), jnp.float32),
            jax.ShapeDtypeStruct((N, 1), jnp.float32),
        ],
    )(degp_t, x)

    zp = _agg_kernel(y, src3, dst3)

    W3p = jnp.zeros((H2, 128), jnp.float32).at[:, :C].set(W3)
    b3p = jnp.zeros((1, 128), jnp.float32).at[:, :C].set(b3)

    R = 2000
    out = pl.pallas_call(
        _mlp_body,
        grid=(N // R,),
        in_specs=[
            pl.BlockSpec((NC, R, D), lambda i: (0, i, 0)),
            pl.BlockSpec((R, D), lambda i: (i, 0)),
            pl.BlockSpec((R, 1), lambda i: (i, 0)),
            pl.BlockSpec((D, H1), lambda i: (0, 0)),
            pl.BlockSpec((1, H1), lambda i: (0, 0)),
            pl.BlockSpec((1, H1), lambda i: (0, 0)),
            pl.BlockSpec((1, H1), lambda i: (0, 0)),
            pl.BlockSpec((H1, H2), lambda i: (0, 0)),
            pl.BlockSpec((1, H2), lambda i: (0, 0)),
            pl.BlockSpec((H2, 128), lambda i: (0, 0)),
            pl.BlockSpec((1, 128), lambda i: (0, 0)),
        ],
        out_specs=pl.BlockSpec((R, 128), lambda i: (i, 0)),
        out_shape=jax.ShapeDtypeStruct((N, 128), jnp.float32),
    )(zp, y, dinv, W1, b1.reshape(1, H1), gamma.reshape(1, H1),
      beta.reshape(1, H1), W2, b2.reshape(1, H2), W3p, b3p)
    return out[:, :C]
